# baseline (speedup 1.0000x reference)
import functools

import jax
import jax.numpy as jnp
from jax import lax
from jax.experimental import pallas as pl
from jax.experimental.pallas import tpu as pltpu
from jax.experimental.pallas import tpu_sc as plsc

N_NODES = 10000
N_EDGES = 320000
D = 128

NC = 2
NS = 16
DH = D // NC
EPT = N_EDGES // NS
CH = 80
K = 5
NCH = EPT // CH
NSTG = 10
NCH_S = NCH // NSTG
NBH_S = NCH_S // K
ACC_ROWS = N_NODES
ZPT = 632
LAST = N_NODES - (NS - 1) * ZPT
LANES = 16
CR = 160
PREV_OFF = 400


def _make_mega():
  mesh = plsc.VectorSubcoreMesh(core_axis_name="c", subcore_axis_name="s")

  tab_t = jax.ShapeDtypeStruct((NC, N_NODES, DH), jnp.float32)
  out_type = (tab_t, tab_t, tab_t, tab_t)
  scratch = [
      pltpu.VMEM((NCH_S, CH), jnp.int32),
      pltpu.VMEM((NCH_S, CH), jnp.int32),
      pltpu.VMEM((2 * K * CH, DH), jnp.float32),
      pltpu.VMEM((CR, LANES), jnp.float32),
      pltpu.VMEM((CH, LANES), jnp.float32),
      pltpu.VMEM((2, LANES), jnp.float32),
      pltpu.VMEM_SHARED((ACC_ROWS, DH), jnp.float32),
      pltpu.VMEM_SHARED((ACC_ROWS, LANES), jnp.float32),
      pltpu.VMEM_SHARED((ACC_ROWS, LANES), jnp.float32),
      pltpu.SemaphoreType.DMA,
      pltpu.SemaphoreType.DMA,
      pltpu.SemaphoreType.DMA,
  ]

  @functools.partial(
      pl.kernel, out_type=out_type, mesh=mesh, scratch_types=scratch,
      compiler_params=pltpu.CompilerParams(use_tc_tiling_on_sc=False))
  def mega(a_tab_h, p_tab_h, src_h, dst_h, ones_h, zeros_h, sw_h,
           a1_h, p1_h, a2_h, p2_h,
           srcv, dstv, rows_v, degv, ones_v, swv,
           acc_sh, degd_sh, degs_sh, gsem, ssem, dsem):
    cid = lax.axis_index("c")
    sid = lax.axis_index("s")

    zvec = jnp.zeros((LANES,), jnp.float32)
    base = sid * ZPT

    def zero_acc():
      def zrow(r, carry):
        for j in range(DH // LANES):
          rows_v[r, pl.ds(j * LANES, LANES)] = zvec
        return carry

      lax.fori_loop(0, ZPT, zrow, 0)

      @pl.when(sid < NS - 1)
      def _zfull():
        pltpu.sync_copy(rows_v.at[pl.ds(0, ZPT)],
                        acc_sh.at[pl.ds(base, ZPT)])

      @pl.when(sid == NS - 1)
      def _zlast():
        pltpu.sync_copy(rows_v.at[pl.ds(0, LAST)],
                        acc_sh.at[pl.ds((NS - 1) * ZPT, LAST)])

    def zero_deg(deg_sh):
      pltpu.sync_copy(zeros_h, ones_v)

      @pl.when(sid < NS - 1)
      def _dzfull():
        for rep in range(ZPT // CH):
          pltpu.sync_copy(ones_v, deg_sh.at[pl.ds(base + rep * CH, CH)])
        rem = ZPT - (ZPT // CH) * CH
        pltpu.sync_copy(ones_v.at[pl.ds(0, rem)],
                        deg_sh.at[pl.ds(base + (ZPT // CH) * CH, rem)])

      @pl.when(sid == NS - 1)
      def _dzlast():
        lb = (NS - 1) * ZPT
        for rep in range(LAST // CH):
          pltpu.sync_copy(ones_v, deg_sh.at[pl.ds(lb + rep * CH, CH)])
        rem = LAST - (LAST // CH) * CH
        pltpu.sync_copy(ones_v.at[pl.ds(0, rem)],
                        deg_sh.at[pl.ds(lb + (LAST // CH) * CH, rem)])

    zero_acc()
    zero_deg(degd_sh)
    zero_deg(degs_sh)
    pltpu.sync_copy(ones_h, ones_v)
    pltpu.sync_copy(sw_h, swv)
    plsc.subcore_barrier()

    def burst_phase(gtab_h, gather_by_src, first):
      gidx_v = srcv if gather_by_src else dstv
      sidx_v = dstv if gather_by_src else srcv
      B = K * CH

      def issue_gathers(g, boff):
        for k in range(K):
          pltpu.async_copy(
              gtab_h.at[cid].at[gidx_v.at[g * K + k]],
              rows_v.at[pl.ds(boff + k * CH, CH)], gsem)

      def wait_gathers(g, boff):
        for k in range(K):
          pltpu.make_async_copy(
              gtab_h.at[cid].at[gidx_v.at[g * K + k]],
              rows_v.at[pl.ds(boff + k * CH, CH)], gsem).wait()

      def issue_scatters(g, boff):
        for k in range(K):
          j = g * K + k
          pltpu.async_copy(
              rows_v.at[pl.ds(boff + k * CH, CH)], acc_sh.at[sidx_v.at[j]],
              ssem, add=True)
          if first:
            pltpu.async_copy(ones_v, degd_sh.at[dstv.at[j]], dsem, add=True)
            pltpu.async_copy(ones_v, degs_sh.at[srcv.at[j]], dsem, add=True)

      def wait_scatters(g, boff):
        for k in range(K):
          pltpu.make_async_copy(
              rows_v.at[pl.ds(boff + k * CH, CH)],
              acc_sh.at[sidx_v.at[g * K + k]], ssem).wait()

      def stage_body(stg, carry):
        pltpu.sync_copy(src_h.at[sid, stg], srcv)
        pltpu.sync_copy(dst_h.at[sid, stg], dstv)

        issue_gathers(0, 0)
        wait_gathers(0, 0)
        issue_gathers(1, B)
        issue_scatters(0, 0)

        def body(g, carry2):
          boff = (g % 2) * B
          alt = B - boff
          wait_gathers(g, boff)
          wait_scatters(g - 1, alt)
          issue_gathers(g + 1, alt)
          issue_scatters(g, boff)
          return carry2

        lax.fori_loop(1, NBH_S - 1, body, 0)
        gl = NBH_S - 1
        boff_l = (gl % 2) * B
        wait_gathers(gl, boff_l)
        wait_scatters(gl - 1, B - boff_l)
        issue_scatters(gl, boff_l)
        wait_scatters(gl, boff_l)

        if first:
          def wdeg(j, carry2):
            pltpu.make_async_copy(ones_v, degd_sh.at[dstv.at[j]],
                                  dsem).wait()
            pltpu.make_async_copy(ones_v, degs_sh.at[srcv.at[j]],
                                  dsem).wait()
            return carry2

          lax.fori_loop(0, NCH_S, wdeg, 0)
        return carry

      lax.fori_loop(0, NSTG, stage_body, 0)

    def combine(out_h, prev_h, deg_sh, sw_idx, first_round):
      sw = swv[sw_idx]

      def do_chunk(st, rr):
        pltpu.sync_copy(acc_sh.at[pl.ds(st, rr)], rows_v.at[pl.ds(0, rr)])
        pltpu.sync_copy(prev_h.at[cid].at[pl.ds(st, rr)],
                        rows_v.at[pl.ds(PREV_OFF, rr)])
        pltpu.sync_copy(deg_sh.at[pl.ds(st, rr)], degv.at[pl.ds(0, rr)])

        def row_body(r, carry):
          dvec = degv[r]
          if first_round:
            inv = 1.0 / jnp.maximum(dvec, 1.0)
            degv[r] = inv
          else:
            inv = dvec
          for c4 in range(DH // LANES):
            s = rows_v[r, pl.ds(c4 * LANES, LANES)]
            prv = rows_v[PREV_OFF + r, pl.ds(c4 * LANES, LANES)]
            rows_v[r, pl.ds(c4 * LANES, LANES)] = s * inv + prv * sw
          return carry

        lax.fori_loop(0, rr, row_body, 0)
        pltpu.sync_copy(rows_v.at[pl.ds(0, rr)],
                        out_h.at[cid].at[pl.ds(st, rr)])
        if first_round:
          pltpu.sync_copy(degv.at[pl.ds(0, rr)], deg_sh.at[pl.ds(st, rr)])

      for st in range(0, 480, CR):
        do_chunk(base + st, CR)

      @pl.when(sid < NS - 1)
      def _tail_full():
        do_chunk(base + 480, ZPT - 480)

      @pl.when(sid == NS - 1)
      def _tail_last():
        do_chunk(base + 480, LAST - 480)

    burst_phase(a_tab_h, gather_by_src=True, first=True)
    plsc.subcore_barrier()
    combine(p1_h, p_tab_h, degd_sh, 0, True)
    zero_acc()
    plsc.subcore_barrier()

    burst_phase(p_tab_h, gather_by_src=False, first=False)
    plsc.subcore_barrier()
    combine(a1_h, a_tab_h, degs_sh, 1, True)
    zero_acc()
    plsc.subcore_barrier()

    burst_phase(a1_h, gather_by_src=True, first=False)
    plsc.subcore_barrier()
    combine(p2_h, p1_h, degd_sh, 0, False)
    zero_acc()
    plsc.subcore_barrier()

    burst_phase(p1_h, gather_by_src=False, first=False)
    plsc.subcore_barrier()
    combine(a2_h, a1_h, degs_sh, 1, False)

  return mega


_sc_mega = _make_mega()


def kernel(author_emb, paper_emb, edge_index, u_sw, i_sw):
  src = edge_index[0].reshape(NS, NSTG, NCH_S, CH)
  dst = edge_index[1].reshape(NS, NSTG, NCH_S, CH)
  a_tab = author_emb.reshape(N_NODES, NC, DH).transpose(1, 0, 2)
  p_tab = paper_emb.reshape(N_NODES, NC, DH).transpose(1, 0, 2)
  ones16 = jnp.ones((CH, LANES), jnp.float32)
  zeros16 = jnp.zeros((CH, LANES), jnp.float32)
  sw = jnp.stack([jnp.full((LANES,), i_sw, jnp.float32),
                  jnp.full((LANES,), u_sw, jnp.float32)])

  _, _, a2, p2 = _sc_mega(a_tab, p_tab, src, dst, ones16, zeros16, sw)
  a_out = a2.transpose(1, 0, 2).reshape(N_NODES, D)
  p_out = p2.transpose(1, 0, 2).reshape(N_NODES, D)
  return (a_out, p_out)
